# speedup vs baseline: 1.0033x; 1.0033x over previous
import functools

import jax
import jax.numpy as jnp
from jax.experimental import pallas as pl
from jax.experimental.pallas import tpu as pltpu

N = 4096
D = 128
BM = 256
NB = N // BM
N_LAYERS = 3


def _body(sw_ref, a2_ref, a1_ref, ac_ref, e0_ref, out_ref, e_ref, t1_ref, t_ref):
    l = pl.program_id(0)
    s = pl.program_id(1)
    i = pl.program_id(2)
    r0 = i * BM

    @pl.when((l == 0) & (s == 0) & (i == 0))
    def _init():
        e_ref[...] = e0_ref[...].astype(jnp.bfloat16)
        out_ref[...] = e0_ref[...] * 0.25

    @pl.when(s == 0)
    def _s0():
        blk = a2_ref[...].astype(jnp.bfloat16)
        t1_ref[pl.ds(r0, BM), :] = jnp.dot(
            blk, e_ref[...], preferred_element_type=jnp.float32
        ).astype(jnp.bfloat16)

    @pl.when(s == 1)
    def _s1():
        blk = a1_ref[...].astype(jnp.bfloat16)
        t_ref[pl.ds(r0, BM), :] = jnp.dot(
            blk, t1_ref[...], preferred_element_type=jnp.float32
        ).astype(jnp.bfloat16)

    @pl.when(s == 2)
    def _s2():
        blk = ac_ref[...].astype(jnp.bfloat16)
        tc = jnp.dot(blk, t_ref[...], preferred_element_type=jnp.float32)
        sl = sw_ref[l]
        tloc = t_ref[pl.ds(r0, BM), :].astype(jnp.float32)
        comb = sl * tc + (1.0 - sl) * tloc
        out_ref[pl.ds(r0, BM), :] += 0.25 * comb

        @pl.when(l < N_LAYERS - 1)
        def _():
            e_ref[pl.ds(r0, BM), :] = comb.astype(jnp.bfloat16)


@functools.partial(jax.jit, static_argnames=("interpret",))
def _side(a2, a1, acat, e0, sw, interpret=False):
    return pl.pallas_call(
        _body,
        grid=(N_LAYERS, 3, NB),
        in_specs=[
            pl.BlockSpec(memory_space=pltpu.SMEM),
            pl.BlockSpec((BM, N), lambda l, s, i: (jnp.where(s == 0, i, NB - 1), 0)),
            pl.BlockSpec(
                (BM, N),
                lambda l, s, i: (jnp.where(s == 1, i, jnp.where(s == 0, 0, NB - 1)), 0),
            ),
            pl.BlockSpec((BM, N), lambda l, s, i: (jnp.where(s == 2, i, 0), 0)),
            pl.BlockSpec((N, D), lambda l, s, i: (0, 0)),
        ],
        out_specs=pl.BlockSpec((N, D), lambda l, s, i: (0, 0)),
        out_shape=jax.ShapeDtypeStruct((N, D), jnp.float32),
        scratch_shapes=[
            pltpu.VMEM((N, D), jnp.bfloat16),
            pltpu.VMEM((N, D), jnp.bfloat16),
            pltpu.VMEM((N, D), jnp.bfloat16),
        ],
        interpret=interpret,
    )(sw, a2, a1, acat, e0)


def kernel(adj_u1, adj_u2, adj_i1, adj_i2, adj_cat, adj_cat_user,
           user_emb, item_emb, scale_weights, interpret=False):
    u_emb = _side(adj_u2, adj_u1, adj_cat_user, user_emb, scale_weights,
                  interpret=interpret)
    i_emb = _side(adj_i2, adj_i1, adj_cat, item_emb, scale_weights,
                  interpret=interpret)
    return (u_emb, i_emb)
